# baseline (speedup 1.0000x reference)
import functools

import jax
import jax.numpy as jnp
from jax import lax
from jax.experimental import pallas as pl
from jax.experimental.pallas import tpu as pltpu
from jax.experimental.pallas import tpu_sc as plsc

VOCAB = 100000
BOTTLENECK = 128
D_MODEL = 1024
N_TOKENS = 4 * 2048

NUM_CORES = 2
NUM_SUBCORES = 16
NW = NUM_CORES * NUM_SUBCORES

N_SLICES = 2
SLICE_ROWS = N_TOKENS // N_SLICES
PER_W = SLICE_ROWS // NW
ROW_TILE = 2048

_sc_mesh = plsc.VectorSubcoreMesh(core_axis_name="c", subcore_axis_name="s")


@functools.cache
def _make_sc_gather(slice_id):
    @functools.partial(
        pl.kernel,
        mesh=_sc_mesh,
        out_type=jax.ShapeDtypeStruct((SLICE_ROWS, BOTTLENECK), jnp.float32),
        scratch_types=[
            pltpu.VMEM((PER_W,), jnp.int32),
            pltpu.VMEM((PER_W, BOTTLENECK), jnp.float32),
            pltpu.SemaphoreType.DMA,
        ],
    )
    def gather(table_hbm, idx_hbm, out_hbm, idx_v, rows_v, sem):
        wid = lax.axis_index("s") * NUM_CORES + lax.axis_index("c")
        pltpu.sync_copy(idx_hbm.at[slice_id, wid], idx_v)
        pltpu.async_copy(table_hbm.at[idx_v], rows_v, sem).wait()
        pltpu.sync_copy(rows_v, out_hbm.at[pl.ds(wid * PER_W, PER_W)])

    return gather


def _mm_first_body(low_ref, w_ref, out_ref):
    out_ref[...] = jnp.dot(
        low_ref[...], w_ref[...], preferred_element_type=jnp.float32
    )


def _mm_chain_body(low_ref, w_ref, acc_ref, out_ref):
    del acc_ref
    out_ref[...] = jnp.dot(
        low_ref[...], w_ref[...], preferred_element_type=jnp.float32
    )


def _mm_slice(low, W, acc, row_off):
    tile_off = row_off // ROW_TILE
    out_spec = pl.BlockSpec(
        (ROW_TILE, D_MODEL), lambda i, _o=tile_off: (i + _o, 0)
    )
    in_specs = [
        pl.BlockSpec((ROW_TILE, BOTTLENECK), lambda i: (i, 0)),
        pl.BlockSpec((BOTTLENECK, D_MODEL), lambda i: (0, 0)),
    ]
    out_shape = jax.ShapeDtypeStruct((N_TOKENS, D_MODEL), jnp.float32)
    if acc is None:
        return pl.pallas_call(
            _mm_first_body,
            grid=(SLICE_ROWS // ROW_TILE,),
            in_specs=in_specs,
            out_specs=out_spec,
            out_shape=out_shape,
        )(low, W)
    return pl.pallas_call(
        _mm_chain_body,
        grid=(SLICE_ROWS // ROW_TILE,),
        in_specs=in_specs + [pl.BlockSpec(memory_space=pl.ANY)],
        out_specs=out_spec,
        out_shape=out_shape,
        input_output_aliases={2: 0},
    )(low, W, acc)


@jax.jit
def kernel(x, embed_table, W):
    idx = x.astype(jnp.int32).reshape(N_SLICES, NW, PER_W)
    lows = [
        _make_sc_gather(k)(embed_table, idx) for k in range(N_SLICES)
    ]
    acc = None
    for k in range(N_SLICES):
        acc = _mm_slice(lows[k], W, acc, k * SLICE_ROWS)
    return acc.reshape(x.shape[0], x.shape[1], D_MODEL)


# speedup vs baseline: 1.0696x; 1.0696x over previous
import functools

import jax
import jax.numpy as jnp
from jax import lax
from jax.experimental import pallas as pl
from jax.experimental.pallas import tpu as pltpu
from jax.experimental.pallas import tpu_sc as plsc

VOCAB = 100000
BOTTLENECK = 128
D_MODEL = 1024
N_TOKENS = 4 * 2048

NUM_CORES = 2
NUM_SUBCORES = 16
NW = NUM_CORES * NUM_SUBCORES
B_PER_W = N_TOKENS // NW
CHUNK = 64
NCHUNK = B_PER_W // CHUNK

_sc_mesh = plsc.VectorSubcoreMesh(core_axis_name="c", subcore_axis_name="s")


@functools.partial(
    pl.kernel,
    mesh=_sc_mesh,
    out_type=jax.ShapeDtypeStruct((N_TOKENS, BOTTLENECK), jnp.float32),
    scratch_types=[
        pltpu.VMEM((NCHUNK, CHUNK), jnp.int32),
        pltpu.VMEM((B_PER_W, BOTTLENECK), jnp.float32),
        pltpu.SemaphoreType.DMA,
    ],
)
def _sc_gather(table_hbm, idx_hbm, out_hbm, idx_v, rows_v, sem):
    wid = lax.axis_index("s") * NUM_CORES + lax.axis_index("c")
    base = wid * B_PER_W
    pltpu.sync_copy(idx_hbm.at[wid], idx_v)
    copies = []
    for j in range(NCHUNK):
        copies.append(
            pltpu.async_copy(
                table_hbm.at[idx_v.at[j]],
                rows_v.at[pl.ds(j * CHUNK, CHUNK)],
                sem,
            )
        )
    for c in copies:
        c.wait()
    pltpu.sync_copy(rows_v, out_hbm.at[pl.ds(base, B_PER_W)])


def _mm_body(low_ref, w_ref, out_ref):
    out_ref[...] = jnp.dot(
        low_ref[...], w_ref[...], preferred_element_type=jnp.float32
    )


ROW_TILE = 2048


@jax.jit
def kernel(x, embed_table, W):
    idx = x.astype(jnp.int32).reshape(NW, NCHUNK, CHUNK)
    low = _sc_gather(embed_table, idx)
    out = pl.pallas_call(
        _mm_body,
        grid=(N_TOKENS // ROW_TILE,),
        in_specs=[
            pl.BlockSpec((ROW_TILE, BOTTLENECK), lambda i: (i, 0)),
            pl.BlockSpec((BOTTLENECK, D_MODEL), lambda i: (0, 0)),
        ],
        out_specs=pl.BlockSpec((ROW_TILE, D_MODEL), lambda i: (i, 0)),
        out_shape=jax.ShapeDtypeStruct((N_TOKENS, D_MODEL), jnp.float32),
    )(low, W)
    return out.reshape(x.shape[0], x.shape[1], D_MODEL)
